# Initial kernel scaffold; baseline (speedup 1.0000x reference)
import jax
import jax.numpy as jnp
from jax.experimental import pallas as pl


def kernel(out, coords_neighbors_idx, n_batch, n_grid, n_ao, W2, b2, W3, b3):
    raise NotImplementedError("write your pallas kernel here")


# speedup vs baseline: 10.6865x; 10.6865x over previous
import jax
import jax.numpy as jnp
from jax.experimental import pallas as pl


def _fused_body(out_ref, idx_ref, w2_ref, b2_ref, w3_ref, b3_ref, o_ref):
    out = out_ref[...]
    B, G, A = out.shape
    S = jnp.sum(out, axis=1)
    M = jax.lax.dot_general(
        S, w2_ref[...], (((1,), (0,)), ((), ())),
        preferred_element_type=jnp.float32,
        precision=jax.lax.Precision.HIGHEST) + b2_ref[...]
    M3 = jax.lax.dot_general(
        M, w3_ref[...], (((1,), (0,)), ((), ())),
        preferred_element_type=jnp.float32,
        precision=jax.lax.Precision.HIGHEST)
    idx = idx_ref[...]
    iota = jax.lax.broadcasted_iota(jnp.int32, (G, B), 1)
    onehot = (idx == iota).astype(jnp.float32)
    msg = jax.lax.dot_general(
        onehot, M3, (((1,), (0,)), ((), ())),
        preferred_element_type=jnp.float32,
        precision=jax.lax.Precision.HIGHEST) + b3_ref[...]
    for b in range(B):
        o_ref[b] = jax.lax.dot_general(
            out[b], w3_ref[...], (((1,), (0,)), ((), ())),
            preferred_element_type=jnp.float32,
            precision=jax.lax.Precision.HIGHEST) + msg


def kernel(out, coords_neighbors_idx, n_batch, n_grid, n_ao, W2, b2, W3, b3):
    B, G, A = out.shape
    idx2d = coords_neighbors_idx.astype(jnp.int32).reshape(G, 1)
    return pl.pallas_call(
        _fused_body,
        out_shape=jax.ShapeDtypeStruct((B, G, A), jnp.float32),
    )(out, idx2d, W2, b2.reshape(1, A), W3, b3.reshape(1, A))
